# baseline (speedup 1.0000x reference)
import jax
import jax.numpy as jnp
from jax.experimental import pallas as pl
from jax.experimental.pallas import tpu as pltpu

NN = 10000
BF = jnp.bfloat16


def _bf16_dot(a, b):
    return jnp.dot(a.astype(BF), b.astype(BF),
                   preferred_element_type=jnp.float32)


def _layer1_body(x_ref, w1_ref, w2_ref, adj_ref, abf_ref, h1_ref, y2_ref, y1_s):
    @pl.when(pl.program_id(0) == 0)
    def _():
        y1_s[...] = _bf16_dot(x_ref[...], w1_ref[...]).astype(BF)

    ab = adj_ref[...].astype(BF)
    abf_ref[...] = ab
    z = jnp.maximum(
        jnp.dot(ab, y1_s[...], preferred_element_type=jnp.float32), 0.0)
    h1_ref[...] = z
    y2_ref[...] = _bf16_dot(z, w2_ref[...]).astype(BF)


def _layer1(x, adj, w1, w2):
    bm = 400
    e_in, e_out = w1.shape[1], w2.shape[1]
    return pl.pallas_call(
        _layer1_body,
        grid=(NN // bm,),
        in_specs=[
            pl.BlockSpec((NN, w1.shape[0]), lambda i: (0, 0)),
            pl.BlockSpec(w1.shape, lambda i: (0, 0)),
            pl.BlockSpec(w2.shape, lambda i: (0, 0)),
            pl.BlockSpec((bm, NN), lambda i: (i, 0)),
        ],
        out_specs=[
            pl.BlockSpec((bm, NN), lambda i: (i, 0)),
            pl.BlockSpec((bm, e_in), lambda i: (i, 0)),
            pl.BlockSpec((bm, e_out), lambda i: (i, 0)),
        ],
        out_shape=[
            jax.ShapeDtypeStruct((NN, NN), BF),
            jax.ShapeDtypeStruct((NN, e_in), jnp.float32),
            jax.ShapeDtypeStruct((NN, e_out), BF),
        ],
        scratch_shapes=[pltpu.VMEM((NN, e_in), BF)],
    )(x, w1, w2, adj)


BMM = 1000
NBM = NN // BMM


def _mids_body(abf_ref, y2_ref, w3_ref, w4_ref,
               h2_ref, h3_ref, h4_ref, h_ref, hbf_ref,
               y3_s, y4_s):
    g = pl.program_id(0)
    l = g // NBM
    i = g % NBM

    @pl.when(l == 0)
    def _():
        z = jnp.maximum(
            jnp.dot(abf_ref[...], y2_ref[...],
                    preferred_element_type=jnp.float32), 0.0)
        h2_ref[...] = z
        y3_s[pl.ds(i * BMM, BMM)] = _bf16_dot(z, w3_ref[...])

    @pl.when(l == 1)
    def _():
        z = jnp.maximum(
            jnp.dot(abf_ref[...], y3_s[pl.ds(0, NN)].astype(BF),
                    preferred_element_type=jnp.float32), 0.0)
        h3_ref[...] = z
        y4_s[pl.ds(i * BMM, BMM)] = _bf16_dot(z, w4_ref[...])

    @pl.when(l == 2)
    def _():
        z = jnp.maximum(
            jnp.dot(abf_ref[...], y4_s[pl.ds(0, NN)].astype(BF),
                    preferred_element_type=jnp.float32), 0.0)
        h4_ref[...] = z
        n = jnp.maximum(jnp.sqrt(jnp.sum(z * z, axis=1, keepdims=True)), 1e-12)
        h = z / n
        h_ref[...] = h
        hbf_ref[...] = h.astype(BF)


def _mids(abf, y2, w3, w4):
    e2, e3, e4 = y2.shape[1], w3.shape[1], w4.shape[1]
    return pl.pallas_call(
        _mids_body,
        grid=(3 * NBM,),
        in_specs=[
            pl.BlockSpec((BMM, NN), lambda g: (g % NBM, 0)),
            pl.BlockSpec((NN, e2), lambda g: (0, 0)),
            pl.BlockSpec(w3.shape, lambda g: (0, 0)),
            pl.BlockSpec(w4.shape, lambda g: (0, 0)),
        ],
        out_specs=[
            pl.BlockSpec((BMM, e2), lambda g: (jnp.clip(g, 0, NBM - 1), 0)),
            pl.BlockSpec((BMM, e3),
                         lambda g: (jnp.clip(g - NBM, 0, NBM - 1), 0)),
            pl.BlockSpec((BMM, e4),
                         lambda g: (jnp.clip(g - 2 * NBM, 0, NBM - 1), 0)),
            pl.BlockSpec((BMM, e4),
                         lambda g: (jnp.clip(g - 2 * NBM, 0, NBM - 1), 0)),
            pl.BlockSpec((BMM, e4),
                         lambda g: (jnp.clip(g - 2 * NBM, 0, NBM - 1), 0)),
        ],
        out_shape=[
            jax.ShapeDtypeStruct((NN, e2), jnp.float32),
            jax.ShapeDtypeStruct((NN, e3), jnp.float32),
            jax.ShapeDtypeStruct((NN, e4), jnp.float32),
            jax.ShapeDtypeStruct((NN, e4), jnp.float32),
            jax.ShapeDtypeStruct((NN, e4), BF),
        ],
        scratch_shapes=[
            pltpu.VMEM((NN, e3), jnp.float32),
            pltpu.VMEM((NN, e4), jnp.float32),
        ],
    )(abf, y2, w3, w4)


def _ahat_body(hblk_ref, hfull_ref, out_ref):
    t = jax.lax.dot_general(
        hblk_ref[...], hfull_ref[...],
        (((1,), (1,)), ((), ())),
        preferred_element_type=jnp.float32,
    )
    out_ref[...] = 0.5 * jnp.tanh(0.5 * t) + 0.5


def _ahat(hbf):
    bm = 400
    e = hbf.shape[1]
    return pl.pallas_call(
        _ahat_body,
        grid=(NN // bm,),
        in_specs=[
            pl.BlockSpec((bm, e), lambda i: (i, 0)),
            pl.BlockSpec((NN, e), lambda i: (0, 0)),
        ],
        out_specs=pl.BlockSpec((bm, NN), lambda i: (i, 0)),
        out_shape=jax.ShapeDtypeStruct((NN, NN), jnp.float32),
    )(hbf, hbf)


def kernel(x, adj, W1, W2, W3, W4):
    adj_bf, enc_h1, y2 = _layer1(x, adj, W1, W2)
    enc_h2, enc_h3, enc_h4, h, hbf = _mids(adj_bf, y2, W3, W4)
    a_hat = _ahat(hbf)
    return (enc_h1, enc_h2, enc_h3, enc_h4, h, a_hat)


# speedup vs baseline: 1.0159x; 1.0159x over previous
import jax
import jax.numpy as jnp
from jax.experimental import pallas as pl
from jax.experimental.pallas import tpu as pltpu

NN = 10000
BF = jnp.bfloat16


def _bf16_dot(a, b):
    return jnp.dot(a.astype(BF), b.astype(BF),
                   preferred_element_type=jnp.float32)


def _layer1_body(x_ref, w1_ref, w2_ref, adj_ref, abf_ref, h1_ref, y2_ref, y1_s):
    @pl.when(pl.program_id(0) == 0)
    def _():
        y1_s[...] = _bf16_dot(x_ref[...], w1_ref[...]).astype(BF)

    ab = adj_ref[...].astype(BF)
    abf_ref[...] = ab
    z = jnp.maximum(
        jnp.dot(ab, y1_s[...], preferred_element_type=jnp.float32), 0.0)
    h1_ref[...] = z
    y2_ref[...] = _bf16_dot(z, w2_ref[...]).astype(BF)


def _layer1(x, adj, w1, w2):
    bm = 400
    e_in, e_out = w1.shape[1], w2.shape[1]
    return pl.pallas_call(
        _layer1_body,
        grid=(NN // bm,),
        in_specs=[
            pl.BlockSpec((NN, w1.shape[0]), lambda i: (0, 0)),
            pl.BlockSpec(w1.shape, lambda i: (0, 0)),
            pl.BlockSpec(w2.shape, lambda i: (0, 0)),
            pl.BlockSpec((bm, NN), lambda i: (i, 0)),
        ],
        out_specs=[
            pl.BlockSpec((bm, NN), lambda i: (i, 0)),
            pl.BlockSpec((bm, e_in), lambda i: (i, 0)),
            pl.BlockSpec((bm, e_out), lambda i: (i, 0)),
        ],
        out_shape=[
            jax.ShapeDtypeStruct((NN, NN), BF),
            jax.ShapeDtypeStruct((NN, e_in), jnp.float32),
            jax.ShapeDtypeStruct((NN, e_out), BF),
        ],
        scratch_shapes=[pltpu.VMEM((NN, e_in), BF)],
    )(x, w1, w2, adj)


BMM = 1000
NBM = NN // BMM


def _mids_body(abf_ref, y2_ref, w3_ref, w4_ref,
               h2_ref, h3_ref, h4_ref, h_ref, hbf_ref,
               y3_s, y4_s):
    g = pl.program_id(0)
    l = g // NBM
    i = g % NBM

    @pl.when(l == 0)
    def _():
        z = jnp.maximum(
            jnp.dot(abf_ref[...], y2_ref[...],
                    preferred_element_type=jnp.float32), 0.0)
        h2_ref[...] = z
        y3_s[pl.ds(i * BMM, BMM)] = _bf16_dot(z, w3_ref[...])

    @pl.when(l == 1)
    def _():
        z = jnp.maximum(
            jnp.dot(abf_ref[...], y3_s[pl.ds(0, NN)].astype(BF),
                    preferred_element_type=jnp.float32), 0.0)
        h3_ref[...] = z
        y4_s[pl.ds(i * BMM, BMM)] = _bf16_dot(z, w4_ref[...])

    @pl.when(l == 2)
    def _():
        z = jnp.maximum(
            jnp.dot(abf_ref[...], y4_s[pl.ds(0, NN)].astype(BF),
                    preferred_element_type=jnp.float32), 0.0)
        h4_ref[...] = z
        n = jnp.maximum(jnp.sqrt(jnp.sum(z * z, axis=1, keepdims=True)), 1e-12)
        h = z / n
        h_ref[...] = h
        hbf_ref[...] = h.astype(BF)


def _mids(abf, y2, w3, w4):
    e2, e3, e4 = y2.shape[1], w3.shape[1], w4.shape[1]
    return pl.pallas_call(
        _mids_body,
        grid=(3 * NBM,),
        in_specs=[
            pl.BlockSpec((BMM, NN), lambda g: (g % NBM, 0)),
            pl.BlockSpec((NN, e2), lambda g: (0, 0)),
            pl.BlockSpec(w3.shape, lambda g: (0, 0)),
            pl.BlockSpec(w4.shape, lambda g: (0, 0)),
        ],
        out_specs=[
            pl.BlockSpec((BMM, e2), lambda g: (jnp.clip(g, 0, NBM - 1), 0)),
            pl.BlockSpec((BMM, e3),
                         lambda g: (jnp.clip(g - NBM, 0, NBM - 1), 0)),
            pl.BlockSpec((BMM, e4),
                         lambda g: (jnp.clip(g - 2 * NBM, 0, NBM - 1), 0)),
            pl.BlockSpec((BMM, e4),
                         lambda g: (jnp.clip(g - 2 * NBM, 0, NBM - 1), 0)),
            pl.BlockSpec((BMM, e4),
                         lambda g: (jnp.clip(g - 2 * NBM, 0, NBM - 1), 0)),
        ],
        out_shape=[
            jax.ShapeDtypeStruct((NN, e2), jnp.float32),
            jax.ShapeDtypeStruct((NN, e3), jnp.float32),
            jax.ShapeDtypeStruct((NN, e4), jnp.float32),
            jax.ShapeDtypeStruct((NN, e4), jnp.float32),
            jax.ShapeDtypeStruct((NN, e4), BF),
        ],
        scratch_shapes=[
            pltpu.VMEM((NN, e3), jnp.float32),
            pltpu.VMEM((NN, e4), jnp.float32),
        ],
    )(abf, y2, w3, w4)


def _ahat_body(hblk_ref, hfull_ref, out_ref):
    t = jax.lax.dot_general(
        hblk_ref[...], hfull_ref[...],
        (((1,), (1,)), ((), ())),
        preferred_element_type=jnp.float32,
    )
    out_ref[...] = 0.5 * jnp.tanh(0.5 * t) + 0.5


def _ahat(hbf):
    bm = 200
    e = hbf.shape[1]
    return pl.pallas_call(
        _ahat_body,
        grid=(NN // bm,),
        in_specs=[
            pl.BlockSpec((bm, e), lambda i: (i, 0)),
            pl.BlockSpec((NN, e), lambda i: (0, 0)),
        ],
        out_specs=pl.BlockSpec((bm, NN), lambda i: (i, 0)),
        out_shape=jax.ShapeDtypeStruct((NN, NN), jnp.float32),
    )(hbf, hbf)


def kernel(x, adj, W1, W2, W3, W4):
    adj_bf, enc_h1, y2 = _layer1(x, adj, W1, W2)
    enc_h2, enc_h3, enc_h4, h, hbf = _mids(adj_bf, y2, W3, W4)
    a_hat = _ahat(hbf)
    return (enc_h1, enc_h2, enc_h3, enc_h4, h, a_hat)
